# baseline (speedup 1.0000x reference)
import functools

import jax
import jax.numpy as jnp
from jax import lax
from jax.experimental import pallas as pl
from jax.experimental.pallas import tpu as pltpu
from jax.experimental.pallas import tpu_sc as plsc

N = 10000
E = 320000
D_IN = 128
D_MODEL = 256
H = D_MODEL // 2

NC = 2
NS = 16
NW = NC * NS

NP = 10240
ECW = E // NW
B = 80
EPAD = 327680
EPB = EPAD // B
PADV = 10200
NBLK = EPB // NS
R = NP // NS
RCH = 128
BN = 1000

_mesh = plsc.VectorSubcoreMesh(core_axis_name="c", subcore_axis_name="s")


HR = NP // 128


def _hist_body(ei_flat_hbm, out_hbm, dst_v, hist_v, idx_v, deg_sh):
    c = lax.axis_index("c")
    s = lax.axis_index("s")
    wid = s * NC + c

    def zero(i, _):
        r = i >> 3
        j = i & 7
        hist_v[r, pl.ds(j * 16, 16)] = jnp.zeros((16,), jnp.float32)
        return 0

    lax.fori_loop(0, HR * 8, zero, 0)

    for j in range(HR // 16):
        idx_v[pl.ds(j * 16, 16)] = lax.iota(jnp.int32, 16) + (j * 16)

    @pl.when(s == 0)
    def _():
        pltpu.sync_copy(hist_v, deg_sh)

    pltpu.sync_copy(ei_flat_hbm.at[pl.ds(E + wid * ECW, ECW)], dst_v)
    plsc.subcore_barrier()

    ones = jnp.full((16,), 1.0, jnp.float32)

    def add(i, _):
        d = dst_v[pl.ds(pl.multiple_of(i * 16, 16), 16)]
        plsc.addupdate_scatter(hist_v, [d >> 7, d & 127], ones)
        return 0

    lax.fori_loop(0, ECW // 16, add, 0)
    pltpu.sync_copy(hist_v, deg_sh.at[idx_v], add=True)
    plsc.subcore_barrier()

    @pl.when(s < 5)
    def _():
        sl = pl.ds(s * 16, 16)
        pltpu.sync_copy(deg_sh.at[sl], out_hbm.at[c, sl])


_k1 = functools.partial(
    pl.kernel,
    out_type=jax.ShapeDtypeStruct((NC, HR, 128), jnp.float32),
    mesh=_mesh,
    scratch_types=[
        pltpu.VMEM((ECW,), jnp.int32),
        pltpu.VMEM((HR, 128), jnp.float32),
        pltpu.VMEM((HR,), jnp.int32),
        pltpu.VMEM_SHARED((HR, 128), jnp.float32),
    ],
    compiler_params=pltpu.CompilerParams(needs_layout_passes=False),
)(_hist_body)


def _mma_body(x_ref, w_ref, h_ref):
    h_ref[...] = jnp.dot(x_ref[...], w_ref[...],
                         preferred_element_type=jnp.float32)


def _k2a(x, W):
    return pl.pallas_call(
        _mma_body,
        grid=(N // BN,),
        in_specs=[
            pl.BlockSpec((BN, D_IN), lambda i: (i, 0)),
            pl.BlockSpec((D_IN, D_MODEL), lambda i: (0, 0)),
        ],
        out_specs=pl.BlockSpec((BN, D_MODEL), lambda i: (i, 0)),
        out_shape=jax.ShapeDtypeStruct((N, D_MODEL), jnp.float32),
    )(x, W)


def _mmb_body(h_ref, part_ref, g0_ref, g1_ref, dinv_ref):
    deg = jnp.sum(part_ref[...], axis=1) + 1.0
    dinv = lax.rsqrt(deg)
    g = h_ref[...] * dinv[:, None]
    g0_ref[...] = g[:, :H]
    g1_ref[...] = g[:, H:]
    dinv_ref[...] = dinv[:, None]


def _k2b(h, part):
    return pl.pallas_call(
        _mmb_body,
        grid=(N // BN,),
        in_specs=[
            pl.BlockSpec((BN, D_MODEL), lambda i: (i, 0)),
            pl.BlockSpec((BN, NC), lambda i: (i, 0)),
        ],
        out_specs=[
            pl.BlockSpec((BN, H), lambda i: (i, 0)),
            pl.BlockSpec((BN, H), lambda i: (i, 0)),
            pl.BlockSpec((BN, 1), lambda i: (i, 0)),
        ],
        out_shape=[
            jax.ShapeDtypeStruct((NP, H), jnp.float32),
            jax.ShapeDtypeStruct((NP, H), jnp.float32),
            jax.ShapeDtypeStruct((N, 1), jnp.float32),
        ],
    )(h, part)


def _scatter_body(src2_hbm, dst2_hbm, g0_hbm, g1_hbm, out0_hbm, out1_hbm,
                  so0, so1, do0, do1,
                  r0, r1, r2, r3,
                  s_sh,
                  os0, os1,
                  gs0, gs1, gs2, gs3,
                  ss0, ss1, ss2, ss3):
    c = lax.axis_index("c")
    s = lax.axis_index("s")
    base = s * NBLK
    so = [so0, so1]
    do = [do0, do1]
    rows = [r0, r1, r2, r3]
    osem = [os0, os1]
    gsem = [gs0, gs1, gs2, gs3]
    ssem = [ss0, ss1, ss2, ss3]

    def fetch_octet(k, p):
        row = base + k
        pltpu.async_copy(src2_hbm.at[pl.ds(row, 8)], so[p], osem[p])
        pltpu.async_copy(dst2_hbm.at[pl.ds(row, 8)], do[p], osem[p])

    def wait_octet(p):
        pltpu.make_async_copy(src2_hbm.at[pl.ds(0, 8)], so[p], osem[p]).wait()
        pltpu.make_async_copy(src2_hbm.at[pl.ds(0, 8)], do[p], osem[p]).wait()

    def start_gather(j):
        p, r = (j % 16) // 8, j % 8

        @pl.when(c == 0)
        def _():
            pltpu.async_copy(g0_hbm.at[so[p].at[r]], rows[j % 4], gsem[j % 4])

        @pl.when(c == 1)
        def _():
            pltpu.async_copy(g1_hbm.at[so[p].at[r]], rows[j % 4], gsem[j % 4])

    def wait_gather(j):
        pltpu.make_async_copy(
            g0_hbm.at[so[0].at[0]], rows[j % 4], gsem[j % 4]).wait()

    def start_scat(j):
        p, r = (j % 16) // 8, j % 8
        pltpu.async_copy(rows[j % 4], s_sh.at[do[p].at[r]], ssem[j % 4],
                         add=True)

    def wait_scat(j):
        p, r = (j % 16) // 8, j % 8
        pltpu.make_async_copy(
            rows[j % 4], s_sh.at[do[p].at[r]], ssem[j % 4]).wait()

    def step(k, j, do_wait_s=True, do_next=True, fetch1=True, fetch0=True):
        wait_gather(j)
        start_scat(j)
        if do_wait_s:
            wait_scat(j + 2)
        if j == 2 and fetch1:
            fetch_octet(k + 6, 1)
        if j == 6 and fetch1:
            wait_octet(1)
        if j == 10 and fetch0:
            fetch_octet(k + 6, 0)
        if j == 14 and fetch0:
            wait_octet(0)
        if do_next:
            start_gather(j + 2)

    fetch_octet(0, 0)

    def init(i, _):
        q = s * R + i * RCH
        sl = pl.ds(q, RCH)

        @pl.when(c == 0)
        def _():
            pltpu.sync_copy(g0_hbm.at[sl], s_sh.at[sl])

        @pl.when(c == 1)
        def _():
            pltpu.sync_copy(g1_hbm.at[sl], s_sh.at[sl])

        return 0

    lax.fori_loop(0, R // RCH, init, 0)
    plsc.subcore_barrier()

    wait_octet(0)
    start_gather(0)
    start_gather(1)
    for j in range(16):
        step(j, j, do_wait_s=(j >= 2))

    def body(q, _):
        k = q * 16
        for j in range(16):
            step(k + j, j)
        return 0

    lax.fori_loop(1, NBLK // 16 - 1, body, 0)

    kl = NBLK - 16
    for j in range(16):
        step(kl + j, j, do_next=(kl + j + 2 < NBLK), fetch0=False)
    wait_scat(NBLK - 2)
    wait_scat(NBLK - 1)
    plsc.subcore_barrier()

    def wb(i, _):
        q = s * R + i * RCH
        sl = pl.ds(q, RCH)

        @pl.when(c == 0)
        def _():
            pltpu.sync_copy(s_sh.at[sl], out0_hbm.at[sl])

        @pl.when(c == 1)
        def _():
            pltpu.sync_copy(s_sh.at[sl], out1_hbm.at[sl])

        return 0

    lax.fori_loop(0, R // RCH, wb, 0)


_k3 = functools.partial(
    pl.kernel,
    out_type=[
        jax.ShapeDtypeStruct((NP, H), jnp.float32),
        jax.ShapeDtypeStruct((NP, H), jnp.float32),
    ],
    mesh=_mesh,
    scratch_types=(
        [pltpu.VMEM((8, B), jnp.int32) for _ in range(4)]
        + [pltpu.VMEM((B, H), jnp.float32) for _ in range(4)]
        + [pltpu.VMEM_SHARED((NP, H), jnp.float32)]
        + [pltpu.SemaphoreType.DMA for _ in range(10)]
    ),
    compiler_params=pltpu.CompilerParams(needs_layout_passes=False),
)(_scatter_body)


def _ep_body(s0_ref, s1_ref, dinv_ref, b_ref, out_ref, out2_ref):
    m = jnp.concatenate([s0_ref[...], s1_ref[...]], axis=1)
    o = (m * dinv_ref[...] + b_ref[...])[None]
    out_ref[...] = o
    out2_ref[...] = o


def _k4(s0, s1, dinv, b2):
    return pl.pallas_call(
        _ep_body,
        grid=(N // BN,),
        in_specs=[
            pl.BlockSpec((BN, H), lambda i: (i, 0)),
            pl.BlockSpec((BN, H), lambda i: (i, 0)),
            pl.BlockSpec((BN, 1), lambda i: (i, 0)),
            pl.BlockSpec((1, D_MODEL), lambda i: (0, 0)),
        ],
        out_specs=[
            pl.BlockSpec((1, BN, D_MODEL), lambda i: (0, i, 0)),
            pl.BlockSpec((1, BN, D_MODEL), lambda i: (0, i, 0)),
        ],
        out_shape=[
            jax.ShapeDtypeStruct((1, N, D_MODEL), jnp.float32),
            jax.ShapeDtypeStruct((1, N, D_MODEL), jnp.float32),
        ],
    )(s0, s1, dinv, b2)


def kernel(x, edge_index, W, b):
    ei_flat = edge_index.reshape(2 * E)
    h = _k2a(x, W)
    part = _k1(ei_flat)
    part_t = part.reshape(NC, NP)[:, :N].T
    g0, g1, dinv = _k2b(h, part_t)
    pad = jnp.full((EPAD - E,), PADV, jnp.int32)
    src2 = jnp.concatenate([edge_index[0], pad]).reshape(EPB, B)
    dst2 = jnp.concatenate([edge_index[1], pad]).reshape(EPB, B)
    s0, s1 = _k3(src2, dst2, g0, g1)
    out, out2 = _k4(s0, s1, dinv, b.reshape(1, D_MODEL))
    return (out, None, out2)


# speedup vs baseline: 2.3698x; 2.3698x over previous
import functools

import jax
import jax.numpy as jnp
from jax import lax
from jax.experimental import pallas as pl
from jax.experimental.pallas import tpu as pltpu
from jax.experimental.pallas import tpu_sc as plsc

N = 10000
E = 320000
D_IN = 128
D_MODEL = 256
H = D_MODEL // 2

NC = 2
NS = 16
NW = NC * NS

NP = 10240
ECW = E // NW
B = 80
EPAD = 327680
EPB = EPAD // B
NBLK = EPB // NS
R = NP // NS
RCH = 128
BN = 1000

_mesh = plsc.VectorSubcoreMesh(core_axis_name="c", subcore_axis_name="s")


HR = NP // 128


def _hist_body(ei_flat_hbm, out_hbm, dst_v, hist_v, idx_v, deg_sh):
    c = lax.axis_index("c")
    s = lax.axis_index("s")
    wid = s * NC + c

    def zero(i, _):
        r = i >> 3
        j = i & 7
        hist_v[r, pl.ds(j * 16, 16)] = jnp.zeros((16,), jnp.float32)
        return 0

    lax.fori_loop(0, HR * 8, zero, 0)

    for j in range(HR // 16):
        idx_v[pl.ds(j * 16, 16)] = lax.iota(jnp.int32, 16) + (j * 16)

    @pl.when(s == 0)
    def _():
        pltpu.sync_copy(hist_v, deg_sh)

    pltpu.sync_copy(ei_flat_hbm.at[pl.ds(E + wid * ECW, ECW)], dst_v)
    plsc.subcore_barrier()

    ones = jnp.full((16,), 1.0, jnp.float32)

    def add(i, _):
        d = dst_v[pl.ds(pl.multiple_of(i * 16, 16), 16)]
        plsc.addupdate_scatter(hist_v, [d >> 7, d & 127], ones)
        return 0

    lax.fori_loop(0, ECW // 16, add, 0)
    pltpu.sync_copy(hist_v, deg_sh.at[idx_v], add=True)
    plsc.subcore_barrier()

    @pl.when(s < 5)
    def _():
        sl = pl.ds(s * 16, 16)
        pltpu.sync_copy(deg_sh.at[sl], out_hbm.at[c, sl])


_k1 = functools.partial(
    pl.kernel,
    out_type=jax.ShapeDtypeStruct((NC, HR, 128), jnp.float32),
    mesh=_mesh,
    scratch_types=[
        pltpu.VMEM((ECW,), jnp.int32),
        pltpu.VMEM((HR, 128), jnp.float32),
        pltpu.VMEM((HR,), jnp.int32),
        pltpu.VMEM_SHARED((HR, 128), jnp.float32),
    ],
    compiler_params=pltpu.CompilerParams(needs_layout_passes=False),
)(_hist_body)


def _mma_body(x_ref, w_ref, h_ref):
    h_ref[...] = jnp.dot(x_ref[...], w_ref[...],
                         preferred_element_type=jnp.float32)


def _k2a(x, W):
    return pl.pallas_call(
        _mma_body,
        grid=(N // BN,),
        in_specs=[
            pl.BlockSpec((BN, D_IN), lambda i: (i, 0)),
            pl.BlockSpec((D_IN, D_MODEL), lambda i: (0, 0)),
        ],
        out_specs=pl.BlockSpec((BN, D_MODEL), lambda i: (i, 0)),
        out_shape=jax.ShapeDtypeStruct((N, D_MODEL), jnp.float32),
    )(x, W)


def _mmb_body(h_ref, part_ref, g0_ref, g1_ref, dinv_ref):
    deg = jnp.sum(part_ref[...], axis=1) + 1.0
    dinv = lax.rsqrt(deg)
    g = h_ref[...] * dinv[:, None]
    g0_ref[...] = g[:, :H]
    g1_ref[...] = g[:, H:]
    dinv_ref[...] = dinv[:, None]


def _k2b(h, part):
    return pl.pallas_call(
        _mmb_body,
        grid=(N // BN,),
        in_specs=[
            pl.BlockSpec((BN, D_MODEL), lambda i: (i, 0)),
            pl.BlockSpec((BN, NC), lambda i: (i, 0)),
        ],
        out_specs=[
            pl.BlockSpec((BN, H), lambda i: (i, 0)),
            pl.BlockSpec((BN, H), lambda i: (i, 0)),
            pl.BlockSpec((BN, 1), lambda i: (i, 0)),
        ],
        out_shape=[
            jax.ShapeDtypeStruct((NP, H), jnp.float32),
            jax.ShapeDtypeStruct((NP, H), jnp.float32),
            jax.ShapeDtypeStruct((N, 1), jnp.float32),
        ],
    )(h, part)


def _scatter_body(src2_hbm, dst2_hbm, g0_hbm, g1_hbm, out0_hbm, out1_hbm,
                  so0, so1, do0, do1,
                  r0, r1, r2, r3,
                  s_sh,
                  os0, os1,
                  gs0, gs1, gs2, gs3,
                  ss0, ss1, ss2, ss3):
    c = lax.axis_index("c")
    s = lax.axis_index("s")
    base = s * NBLK
    so = [so0, so1]
    do = [do0, do1]
    rows = [r0, r1, r2, r3]
    osem = [os0, os1]
    gsem = [gs0, gs1, gs2, gs3]
    ssem = [ss0, ss1, ss2, ss3]

    def fetch_octet(k, p):
        row = base + k
        pltpu.async_copy(src2_hbm.at[pl.ds(row, 8)], so[p], osem[p])
        pltpu.async_copy(dst2_hbm.at[pl.ds(row, 8)], do[p], osem[p])

    def wait_octet(p):
        pltpu.make_async_copy(src2_hbm.at[pl.ds(0, 8)], so[p], osem[p]).wait()
        pltpu.make_async_copy(src2_hbm.at[pl.ds(0, 8)], do[p], osem[p]).wait()

    def start_gather(j):
        p, r = (j % 16) // 8, j % 8

        @pl.when(c == 0)
        def _():
            pltpu.async_copy(g0_hbm.at[so[p].at[r]], rows[j % 4], gsem[j % 4])

        @pl.when(c == 1)
        def _():
            pltpu.async_copy(g1_hbm.at[so[p].at[r]], rows[j % 4], gsem[j % 4])

    def wait_gather(j):
        pltpu.make_async_copy(
            g0_hbm.at[so[0].at[0]], rows[j % 4], gsem[j % 4]).wait()

    def start_scat(j):
        p, r = (j % 16) // 8, j % 8
        pltpu.async_copy(rows[j % 4], s_sh.at[do[p].at[r]], ssem[j % 4],
                         add=True)

    def wait_scat(j):
        p, r = (j % 16) // 8, j % 8
        pltpu.make_async_copy(
            rows[j % 4], s_sh.at[do[p].at[r]], ssem[j % 4]).wait()

    def step(k, j, do_wait_s=True, do_next=True, fetch1=True, fetch0=True):
        wait_gather(j)
        start_scat(j)
        if do_wait_s:
            wait_scat(j + 2)
        if j == 2 and fetch1:
            fetch_octet(k + 6, 1)
        if j == 6 and fetch1:
            wait_octet(1)
        if j == 10 and fetch0:
            fetch_octet(k + 6, 0)
        if j == 14 and fetch0:
            wait_octet(0)
        if do_next:
            start_gather(j + 2)

    fetch_octet(0, 0)

    def init(i, _):
        q = s * R + i * RCH
        sl = pl.ds(q, RCH)

        @pl.when(c == 0)
        def _():
            pltpu.sync_copy(g0_hbm.at[sl], s_sh.at[sl])

        @pl.when(c == 1)
        def _():
            pltpu.sync_copy(g1_hbm.at[sl], s_sh.at[sl])

        return 0

    lax.fori_loop(0, R // RCH, init, 0)
    plsc.subcore_barrier()

    wait_octet(0)
    start_gather(0)
    start_gather(1)
    for j in range(16):
        step(j, j, do_wait_s=(j >= 2))

    def body(q, _):
        k = q * 16
        for j in range(16):
            step(k + j, j)
        return 0

    lax.fori_loop(1, NBLK // 16 - 1, body, 0)

    kl = NBLK - 16
    for j in range(16):
        step(kl + j, j, do_next=(kl + j + 2 < NBLK), fetch0=False)
    wait_scat(NBLK - 2)
    wait_scat(NBLK - 1)
    plsc.subcore_barrier()

    def wb(i, _):
        q = s * R + i * RCH
        sl = pl.ds(q, RCH)

        @pl.when(c == 0)
        def _():
            pltpu.sync_copy(s_sh.at[sl], out0_hbm.at[sl])

        @pl.when(c == 1)
        def _():
            pltpu.sync_copy(s_sh.at[sl], out1_hbm.at[sl])

        return 0

    lax.fori_loop(0, R // RCH, wb, 0)


_k3 = functools.partial(
    pl.kernel,
    out_type=[
        jax.ShapeDtypeStruct((NP, H), jnp.float32),
        jax.ShapeDtypeStruct((NP, H), jnp.float32),
    ],
    mesh=_mesh,
    scratch_types=(
        [pltpu.VMEM((8, B), jnp.int32) for _ in range(4)]
        + [pltpu.VMEM((B, H), jnp.float32) for _ in range(4)]
        + [pltpu.VMEM_SHARED((NP, H), jnp.float32)]
        + [pltpu.SemaphoreType.DMA for _ in range(10)]
    ),
    compiler_params=pltpu.CompilerParams(needs_layout_passes=False),
)(_scatter_body)


def _ep_body(s0_ref, s1_ref, dinv_ref, b_ref, out_ref, out2_ref):
    m = jnp.concatenate([s0_ref[...], s1_ref[...]], axis=1)
    o = (m * dinv_ref[...] + b_ref[...])[None]
    out_ref[...] = o
    out2_ref[...] = o


def _k4(s0, s1, dinv, b2):
    return pl.pallas_call(
        _ep_body,
        grid=(N // BN,),
        in_specs=[
            pl.BlockSpec((BN, H), lambda i: (i, 0)),
            pl.BlockSpec((BN, H), lambda i: (i, 0)),
            pl.BlockSpec((BN, 1), lambda i: (i, 0)),
            pl.BlockSpec((1, D_MODEL), lambda i: (0, 0)),
        ],
        out_specs=[
            pl.BlockSpec((1, BN, D_MODEL), lambda i: (0, i, 0)),
            pl.BlockSpec((1, BN, D_MODEL), lambda i: (0, i, 0)),
        ],
        out_shape=[
            jax.ShapeDtypeStruct((1, N, D_MODEL), jnp.float32),
            jax.ShapeDtypeStruct((1, N, D_MODEL), jnp.float32),
        ],
    )(s0, s1, dinv, b2)


def kernel(x, edge_index, W, b):
    ei_flat = edge_index.reshape(2 * E)
    h = _k2a(x, W)
    part = _k1(ei_flat)
    part_t = part.reshape(NC, NP)[:, :N].T
    g0, g1, dinv = _k2b(h, part_t)
    pad = N + jnp.arange(EPAD - E, dtype=jnp.int32) % (NP - N)
    src2 = jnp.concatenate([edge_index[0], pad]).reshape(EPB, B)
    dst2 = jnp.concatenate([edge_index[1], pad]).reshape(EPB, B)
    s0, s1 = _k3(src2, dst2, g0, g1)
    out, out2 = _k4(s0, s1, dinv, b.reshape(1, D_MODEL))
    return (out, None, out2)


# speedup vs baseline: 2.5679x; 1.0836x over previous
import functools

import jax
import jax.numpy as jnp
from jax import lax
from jax.experimental import pallas as pl
from jax.experimental.pallas import tpu as pltpu
from jax.experimental.pallas import tpu_sc as plsc

N = 10000
E = 320000
D_IN = 128
D_MODEL = 256
H = D_MODEL // 2

NC = 2
NS = 16
NW = NC * NS

NP = 10240
ECW = E // NW
B = 80
NBLK = E // B // NS
R = NP // NS
RCH = 128
BN = 1000

_mesh = plsc.VectorSubcoreMesh(core_axis_name="c", subcore_axis_name="s")


HR = NP // 128


def _hist_body(ei_flat_hbm, out_hbm, dst_v, hist_v, idx_v, deg_sh):
    c = lax.axis_index("c")
    s = lax.axis_index("s")
    wid = s * NC + c

    def zero(i, _):
        r = i >> 3
        j = i & 7
        hist_v[r, pl.ds(j * 16, 16)] = jnp.zeros((16,), jnp.float32)
        return 0

    lax.fori_loop(0, HR * 8, zero, 0)

    for j in range(HR // 16):
        idx_v[pl.ds(j * 16, 16)] = lax.iota(jnp.int32, 16) + (j * 16)

    @pl.when(s == 0)
    def _():
        pltpu.sync_copy(hist_v, deg_sh)

    pltpu.sync_copy(ei_flat_hbm.at[pl.ds(E + wid * ECW, ECW)], dst_v)
    plsc.subcore_barrier()

    ones = jnp.full((16,), 1.0, jnp.float32)

    def add(i, _):
        d = dst_v[pl.ds(pl.multiple_of(i * 16, 16), 16)]
        plsc.addupdate_scatter(hist_v, [d >> 7, d & 127], ones)
        return 0

    lax.fori_loop(0, ECW // 16, add, 0)
    pltpu.sync_copy(hist_v, deg_sh.at[idx_v], add=True)
    plsc.subcore_barrier()

    @pl.when(s < 5)
    def _():
        sl = pl.ds(s * 16, 16)
        pltpu.sync_copy(deg_sh.at[sl], out_hbm.at[c, sl])


_k1 = functools.partial(
    pl.kernel,
    out_type=jax.ShapeDtypeStruct((NC, HR, 128), jnp.float32),
    mesh=_mesh,
    scratch_types=[
        pltpu.VMEM((ECW,), jnp.int32),
        pltpu.VMEM((HR, 128), jnp.float32),
        pltpu.VMEM((HR,), jnp.int32),
        pltpu.VMEM_SHARED((HR, 128), jnp.float32),
    ],
    compiler_params=pltpu.CompilerParams(needs_layout_passes=False),
)(_hist_body)


def _mma_body(x_ref, w_ref, h_ref):
    h_ref[...] = jnp.dot(x_ref[...], w_ref[...],
                         preferred_element_type=jnp.float32)


def _k2a(x, W):
    return pl.pallas_call(
        _mma_body,
        grid=(N // BN,),
        in_specs=[
            pl.BlockSpec((BN, D_IN), lambda i: (i, 0)),
            pl.BlockSpec((D_IN, D_MODEL), lambda i: (0, 0)),
        ],
        out_specs=pl.BlockSpec((BN, D_MODEL), lambda i: (i, 0)),
        out_shape=jax.ShapeDtypeStruct((N, D_MODEL), jnp.float32),
    )(x, W)


def _mmb_body(h_ref, part_ref, g0_ref, g1_ref, dinv_ref):
    deg = jnp.sum(part_ref[...], axis=1) + 1.0
    dinv = lax.rsqrt(deg)
    g = h_ref[...] * dinv[:, None]
    g0_ref[...] = g[:, :H]
    g1_ref[...] = g[:, H:]
    dinv_ref[...] = dinv[:, None]


def _k2b(h, part):
    return pl.pallas_call(
        _mmb_body,
        grid=(N // BN,),
        in_specs=[
            pl.BlockSpec((BN, D_MODEL), lambda i: (i, 0)),
            pl.BlockSpec((BN, NC), lambda i: (i, 0)),
        ],
        out_specs=[
            pl.BlockSpec((BN, H), lambda i: (i, 0)),
            pl.BlockSpec((BN, H), lambda i: (i, 0)),
            pl.BlockSpec((BN, 1), lambda i: (i, 0)),
        ],
        out_shape=[
            jax.ShapeDtypeStruct((NP, H), jnp.float32),
            jax.ShapeDtypeStruct((NP, H), jnp.float32),
            jax.ShapeDtypeStruct((N, 1), jnp.float32),
        ],
    )(h, part)


def _scatter_body(ei_flat_hbm, g0_hbm, g1_hbm, out0_hbm, out1_hbm,
                  e0, e1, e2, e3, e4, e5, e6, e7,
                  r0, r1, r2, r3,
                  s_sh,
                  is0, is1, is2, is3, is4, is5, is6, is7,
                  gs0, gs1, gs2, gs3,
                  ss0, ss1, ss2, ss3):
    c = lax.axis_index("c")
    s = lax.axis_index("s")
    base = s * NBLK
    e = [e0, e1, e2, e3, e4, e5, e6, e7]
    rows = [r0, r1, r2, r3]
    isem = [is0, is1, is2, is3, is4, is5, is6, is7]
    gsem = [gs0, gs1, gs2, gs3]
    ssem = [ss0, ss1, ss2, ss3]

    def fetch_idx(k, j):
        off = (base + k) * B
        pltpu.async_copy(ei_flat_hbm.at[pl.ds(off, B)], e[j % 8].at[0],
                         isem[j % 8])
        pltpu.async_copy(ei_flat_hbm.at[pl.ds(E + off, B)], e[j % 8].at[1],
                         isem[j % 8])

    def wait_idx(j):
        pltpu.make_async_copy(
            ei_flat_hbm.at[pl.ds(0, B)], e[j % 8].at[0], isem[j % 8]).wait()
        pltpu.make_async_copy(
            ei_flat_hbm.at[pl.ds(0, B)], e[j % 8].at[1], isem[j % 8]).wait()

    def start_gather(j):
        @pl.when(c == 0)
        def _():
            pltpu.async_copy(g0_hbm.at[e[j % 8].at[0]], rows[j % 4], gsem[j % 4])

        @pl.when(c == 1)
        def _():
            pltpu.async_copy(g1_hbm.at[e[j % 8].at[0]], rows[j % 4], gsem[j % 4])

    def wait_gather(j):
        pltpu.make_async_copy(g0_hbm.at[e[0].at[0]], rows[j % 4], gsem[j % 4]).wait()

    def start_scat(j):
        pltpu.async_copy(rows[j % 4], s_sh.at[e[j % 8].at[1]], ssem[j % 4], add=True)

    def wait_scat(j):
        pltpu.make_async_copy(
            rows[j % 4], s_sh.at[e[j % 8].at[1]], ssem[j % 4]).wait()

    def step(k, j, do_wait_s=True, do_next=True, do_fetch=True):
        wait_gather(j)
        start_scat(j)
        if do_wait_s:
            wait_scat(j + 2)
        if do_next:
            wait_idx(j + 2)
            start_gather(j + 2)
        if do_fetch:
            fetch_idx(k + 6, j + 6)

    for j in range(6):
        fetch_idx(j, j)

    def init(i, _):
        q = s * R + i * RCH
        sl = pl.ds(q, RCH)

        @pl.when(c == 0)
        def _():
            pltpu.sync_copy(g0_hbm.at[sl], s_sh.at[sl])

        @pl.when(c == 1)
        def _():
            pltpu.sync_copy(g1_hbm.at[sl], s_sh.at[sl])

        return 0

    lax.fori_loop(0, R // RCH, init, 0)
    plsc.subcore_barrier()

    wait_idx(0)
    start_gather(0)
    wait_idx(1)
    start_gather(1)
    for j in range(8):
        step(j, j, do_wait_s=(j >= 2))

    def body(q, _):
        k = q * 8
        for j in range(8):
            step(k + j, j)
        return 0

    lax.fori_loop(1, (NBLK - 10) // 8, body, 0)

    kl = NBLK - 10
    for j in range(10):
        step(kl + j, j, do_next=(kl + j + 2 < NBLK), do_fetch=(kl + j + 6 < NBLK))
    wait_scat(NBLK - 2)
    wait_scat(NBLK - 1)
    plsc.subcore_barrier()

    def wb(i, _):
        q = s * R + i * RCH
        sl = pl.ds(q, RCH)

        @pl.when(c == 0)
        def _():
            pltpu.sync_copy(s_sh.at[sl], out0_hbm.at[sl])

        @pl.when(c == 1)
        def _():
            pltpu.sync_copy(s_sh.at[sl], out1_hbm.at[sl])

        return 0

    lax.fori_loop(0, R // RCH, wb, 0)


_k3 = functools.partial(
    pl.kernel,
    out_type=[
        jax.ShapeDtypeStruct((NP, H), jnp.float32),
        jax.ShapeDtypeStruct((NP, H), jnp.float32),
    ],
    mesh=_mesh,
    scratch_types=(
        [pltpu.VMEM((2, B), jnp.int32) for _ in range(8)]
        + [pltpu.VMEM((B, H), jnp.float32) for _ in range(4)]
        + [pltpu.VMEM_SHARED((NP, H), jnp.float32)]
        + [pltpu.SemaphoreType.DMA for _ in range(16)]
    ),
    compiler_params=pltpu.CompilerParams(needs_layout_passes=False),
)(_scatter_body)


def _ep_body(s0_ref, s1_ref, dinv_ref, b_ref, out_ref, out2_ref):
    m = jnp.concatenate([s0_ref[...], s1_ref[...]], axis=1)
    o = (m * dinv_ref[...] + b_ref[...])[None]
    out_ref[...] = o
    out2_ref[...] = o


def _k4(s0, s1, dinv, b2):
    return pl.pallas_call(
        _ep_body,
        grid=(N // BN,),
        in_specs=[
            pl.BlockSpec((BN, H), lambda i: (i, 0)),
            pl.BlockSpec((BN, H), lambda i: (i, 0)),
            pl.BlockSpec((BN, 1), lambda i: (i, 0)),
            pl.BlockSpec((1, D_MODEL), lambda i: (0, 0)),
        ],
        out_specs=[
            pl.BlockSpec((1, BN, D_MODEL), lambda i: (0, i, 0)),
            pl.BlockSpec((1, BN, D_MODEL), lambda i: (0, i, 0)),
        ],
        out_shape=[
            jax.ShapeDtypeStruct((1, N, D_MODEL), jnp.float32),
            jax.ShapeDtypeStruct((1, N, D_MODEL), jnp.float32),
        ],
    )(s0, s1, dinv, b2)


def kernel(x, edge_index, W, b):
    ei_flat = edge_index.reshape(2 * E)
    h = _k2a(x, W)
    part = _k1(ei_flat)
    part_t = part.reshape(NC, NP)[:, :N].T
    g0, g1, dinv = _k2b(h, part_t)
    s0, s1 = _k3(ei_flat, g0, g1)
    out, out2 = _k4(s0, s1, dinv, b.reshape(1, D_MODEL))
    return (out, None, out2)
